# Initial kernel scaffold; baseline (speedup 1.0000x reference)
import jax
import jax.numpy as jnp
from jax.experimental import pallas as pl


def kernel(feat, proposals, img_shape, target, W6, b6, W7, b7, Wc, bc, Wb, bb):
    raise NotImplementedError("write your pallas kernel here")


# speedup vs baseline: 11.6232x; 11.6232x over previous
import functools

import jax
import jax.numpy as jnp
from jax import lax
from jax.experimental import pallas as pl

_NC = 21
_POOL = 7
_FC = 1024
_NEG = -1e9


def _mm1_kernel(nk, x_ref, w_ref, b_ref, o_ref):
    k = pl.program_id(1)

    @pl.when(k == 0)
    def _():
        o_ref[...] = jnp.zeros_like(o_ref)

    o_ref[...] += jnp.dot(x_ref[...], w_ref[...], preferred_element_type=jnp.float32)

    @pl.when(k == nk - 1)
    def _():
        o_ref[...] = jnp.maximum(o_ref[...] + b_ref[...], 0.0)


def _head_kernel(h6_ref, w7_ref, b7_ref, wcb_ref, bcb_ref, o_ref):
    h7 = jnp.maximum(
        jnp.dot(h6_ref[...], w7_ref[...], preferred_element_type=jnp.float32)
        + b7_ref[...],
        0.0,
    )
    o_ref[...] = (
        jnp.dot(h7, wcb_ref[...], preferred_element_type=jnp.float32) + bcb_ref[...]
    )


def _nms_kernel(x1_ref, y1_ref, x2_ref, y2_ref, s_ref, kp_ref):
    x1 = x1_ref[...]
    y1 = y1_ref[...]
    x2 = x2_ref[...]
    y2 = y2_ref[...]
    area = jnp.maximum(x2 - x1, 0.0) * jnp.maximum(y2 - y1, 0.0)
    lanes = lax.broadcasted_iota(jnp.int32, x1.shape, 1)

    def body(i, carry):
        kp, s = carry
        m = jnp.max(s, axis=1, keepdims=True)
        hit = s == m
        idx = jnp.min(jnp.where(hit, lanes, x1.shape[1]), axis=1, keepdims=True)
        first = lanes == idx
        ok = m > (_NEG * 0.5)
        kp = jnp.where(first & ok, 1.0, kp)
        bx1 = jnp.sum(jnp.where(first, x1, 0.0), axis=1, keepdims=True)
        by1 = jnp.sum(jnp.where(first, y1, 0.0), axis=1, keepdims=True)
        bx2 = jnp.sum(jnp.where(first, x2, 0.0), axis=1, keepdims=True)
        by2 = jnp.sum(jnp.where(first, y2, 0.0), axis=1, keepdims=True)
        barea = jnp.maximum(bx2 - bx1, 0.0) * jnp.maximum(by2 - by1, 0.0)
        xx1 = jnp.maximum(bx1, x1)
        yy1 = jnp.maximum(by1, y1)
        xx2 = jnp.minimum(bx2, x2)
        yy2 = jnp.minimum(by2, y2)
        inter = jnp.maximum(xx2 - xx1, 0.0) * jnp.maximum(yy2 - yy1, 0.0)
        iou = inter / (barea + area - inter + 1e-9)
        s = jnp.where(((iou > 0.5) & ok) | first, _NEG, s)
        return kp, s

    kp, _ = lax.fori_loop(0, 100, body, (jnp.zeros_like(x1), s_ref[...]))
    kp_ref[...] = kp


def kernel(feat, proposals, img_shape, target, W6, b6, W7, b7, Wc, bc, Wb, bb):
    n = proposals.shape[0]
    C = feat.shape[1]
    FH = feat.shape[2]
    FW = feat.shape[3]

    f0 = jnp.transpose(feat[0], (1, 2, 0))
    b = proposals * (1.0 / 16.0)
    x1 = jnp.round(b[:, 0])
    y1 = jnp.round(b[:, 1])
    x2 = jnp.round(b[:, 2])
    y2 = jnp.round(b[:, 3])
    roi_w = jnp.maximum(x2 - x1 + 1.0, 1.0)
    roi_h = jnp.maximum(y2 - y1 + 1.0, 1.0)
    bwid = roi_w / _POOL
    bhgt = roi_h / _POOL
    offs = jnp.array([0.25, 0.75], dtype=jnp.float32)
    grid = jnp.arange(_POOL, dtype=jnp.float32)[:, None] + offs[None, :]
    px = x1[:, None, None] + bwid[:, None, None] * grid[None]
    py = y1[:, None, None] + bhgt[:, None, None] * grid[None]
    xi = jnp.clip(jnp.floor(px), 0, FW - 1).astype(jnp.int32)
    yi = jnp.clip(jnp.floor(py), 0, FH - 1).astype(jnp.int32)
    g = f0[yi[:, :, :, None, None], xi[:, None, None, :, :]]
    pooled = jnp.max(g, axis=(2, 4))
    x = pooled.reshape(n, _POOL * _POOL * C)

    npad = 1024
    x = jnp.pad(x, ((0, npad - n), (0, 0)))
    W6p = W6.reshape(C, _POOL, _POOL, _FC).transpose(1, 2, 0, 3).reshape(-1, _FC)

    KB = 1792
    nkb = (_POOL * _POOL * C) // KB
    h6 = pl.pallas_call(
        functools.partial(_mm1_kernel, nkb),
        grid=(npad // 128, nkb),
        in_specs=[
            pl.BlockSpec((128, KB), lambda i, k: (i, k)),
            pl.BlockSpec((KB, _FC), lambda i, k: (k, 0)),
            pl.BlockSpec((1, _FC), lambda i, k: (0, 0)),
        ],
        out_specs=pl.BlockSpec((128, _FC), lambda i, k: (i, 0)),
        out_shape=jax.ShapeDtypeStruct((npad, _FC), jnp.float32),
    )(x, W6p, b6[None, :])

    Wcb = jnp.zeros((_FC, 128), jnp.float32)
    Wcb = Wcb.at[:, :_NC].set(Wc).at[:, _NC : _NC + 4 * _NC].set(Wb)
    bcb = jnp.zeros((128,), jnp.float32)
    bcb = bcb.at[:_NC].set(bc).at[_NC : _NC + 4 * _NC].set(bb)
    out = pl.pallas_call(
        _head_kernel,
        out_shape=jax.ShapeDtypeStruct((npad, 128), jnp.float32),
    )(h6, W7, b7[None, :], Wcb, bcb[None, :])
    cls = out[:n, :_NC]
    btf = out[:n, _NC : _NC + 4 * _NC].reshape(n, _NC, 4)

    pw = proposals[:, 2] - proposals[:, 0]
    ph = proposals[:, 3] - proposals[:, 1]
    pcx = proposals[:, 0] + 0.5 * pw
    pcy = proposals[:, 1] + 0.5 * ph
    dx = btf[..., 0]
    dy = btf[..., 1]
    dw = jnp.minimum(btf[..., 2], jnp.log(1000.0 / 16.0))
    dh = jnp.minimum(btf[..., 3], jnp.log(1000.0 / 16.0))
    cx = dx * pw[:, None] + pcx[:, None]
    cy = dy * ph[:, None] + pcy[:, None]
    w = jnp.exp(dw) * pw[:, None]
    h = jnp.exp(dh) * ph[:, None]
    boxes = jnp.stack(
        [cx - 0.5 * w, cy - 0.5 * h, cx + 0.5 * w, cy + 0.5 * h], axis=-1
    )
    boxes = jnp.clip(boxes, 0.0, jnp.asarray(img_shape, jnp.float32))
    scores = jax.nn.softmax(cls, axis=1)

    boxes_f = boxes[:, 1:].reshape(-1, 4)
    scores_f = scores[:, 1:].reshape(-1)
    ws = boxes_f[:, 2] - boxes_f[:, 0]
    hs = boxes_f[:, 3] - boxes_f[:, 1]
    valid = (scores_f > 0.05) & (ws >= 1.0) & (hs >= 1.0)

    bcm = boxes[:, 1:, :].transpose(1, 0, 2)
    scm = jnp.where(valid.reshape(n, _NC - 1).T, scores[:, 1:].T, _NEG)

    def p2(a):
        return jnp.pad(a, ((0, 32 - (_NC - 1)), (0, 1024 - n)))

    sc = jnp.pad(scm, ((0, 32 - (_NC - 1)), (0, 1024 - n)), constant_values=_NEG)
    kp = pl.pallas_call(
        _nms_kernel,
        out_shape=jax.ShapeDtypeStruct((32, 1024), jnp.float32),
    )(p2(bcm[..., 0]), p2(bcm[..., 1]), p2(bcm[..., 2]), p2(bcm[..., 3]), sc)

    keep_f = (kp[: _NC - 1, :n] > 0.5).T.reshape(-1)
    final_scores = jnp.where(keep_f & valid, scores_f, _NEG)
    top_s, top_i = lax.top_k(final_scores, 100)
    labels_f = jnp.broadcast_to(jnp.arange(_NC)[None, :], (n, _NC))[:, 1:].reshape(-1)
    return boxes_f[top_i], top_s, labels_f[top_i]
